# baseline (speedup 1.0000x reference)
import functools

import jax
import jax.numpy as jnp
from jax import lax
from jax.experimental import pallas as pl
from jax.experimental.pallas import tpu as pltpu
from jax.experimental.pallas import tpu_sc as plsc

_LANES = 16
_NUM_WORKERS = 32
_TOTAL_OUT = 2 * 8 * 10000 * 9

_CNT_HI = 45008
_CNT_LO = 44992
assert 16 * (_CNT_HI + _CNT_LO) == _TOTAL_OUT

_CW = 22512
_N_WIN = 2
assert _CW % 16 == 0 and _CW * _N_WIN >= _CNT_HI and _CW <= _CNT_LO


def _sc_body(x_hbm, out_hbm, in_v, out_v):
    wid = lax.axis_index("s") * 2 + lax.axis_index("c")
    is_hi = wid < (_NUM_WORKERS // 2)
    count = jnp.where(is_hi, _CNT_HI, _CNT_LO)
    e0 = jnp.where(
        is_hi,
        wid * _CNT_HI,
        (_NUM_WORKERS // 2) * _CNT_HI + (wid - _NUM_WORKERS // 2) * _CNT_LO,
    )
    e1 = e0 + count

    idx0 = 2 * lax.broadcasted_iota(jnp.int32, (_LANES,), 0)

    for w in range(_N_WIN):
        ws = jnp.minimum(e0 + w * _CW, e1 - _CW)
        pltpu.sync_copy(x_hbm.at[pl.ds(2 * ws, 2 * _CW)], in_v)

        def gather_body(j, _, idx0=idx0):
            vals = plsc.load_gather(in_v, [idx0 + j * (2 * _LANES)])
            out_v[pl.ds(j * _LANES, _LANES)] = vals
            return 0

        lax.fori_loop(0, _CW // _LANES, gather_body, 0)
        pltpu.sync_copy(out_v, out_hbm.at[pl.ds(ws, _CW)])


@jax.jit
def _dense_dilated_sc(x_flat):
    mesh = plsc.VectorSubcoreMesh(core_axis_name="c", subcore_axis_name="s")
    return pl.kernel(
        _sc_body,
        out_type=jax.ShapeDtypeStruct((_TOTAL_OUT,), jnp.int32),
        mesh=mesh,
        scratch_types=[
            pltpu.VMEM((2 * _CW,), jnp.int32),
            pltpu.VMEM((_CW,), jnp.int32),
        ],
        compiler_params=pltpu.CompilerParams(needs_layout_passes=False),
    )(x_flat)


def kernel(edge_index):
    x_flat = edge_index.reshape(-1)
    out_flat = _dense_dilated_sc(x_flat)
    return out_flat.reshape(2, 8, 10000, 9)


# speedup vs baseline: 1.0453x; 1.0453x over previous
import functools

import jax
import jax.numpy as jnp
from jax import lax
from jax.experimental import pallas as pl
from jax.experimental.pallas import tpu as pltpu
from jax.experimental.pallas import tpu_sc as plsc

_LANES = 16
_NUM_WORKERS = 32
_TOTAL_OUT = 2 * 8 * 10000 * 9

_CNT_HI = 45008
_CNT_LO = 44992
assert 16 * (_CNT_HI + _CNT_LO) == _TOTAL_OUT

_CW = 11264
_N_WIN = 4
assert _CW % 16 == 0 and _CW * _N_WIN >= _CNT_HI and _CW <= _CNT_LO


def _sc_body(x_hbm, out_hbm, in0, in1, out0, out1, si0, si1, so0, so1):
    wid = lax.axis_index("s") * 2 + lax.axis_index("c")
    is_hi = wid < (_NUM_WORKERS // 2)
    count = jnp.where(is_hi, _CNT_HI, _CNT_LO)
    e0 = jnp.where(
        is_hi,
        wid * _CNT_HI,
        (_NUM_WORKERS // 2) * _CNT_HI + (wid - _NUM_WORKERS // 2) * _CNT_LO,
    )
    e1 = e0 + count

    ins = (in0, in1)
    outs = (out0, out1)
    sis = (si0, si1)
    sos = (so0, so1)

    def win_start(w):
        return jnp.minimum(e0 + w * _CW, e1 - _CW)

    idx0 = 2 * lax.broadcasted_iota(jnp.int32, (_LANES,), 0)

    in_copies = [None] * _N_WIN
    out_copies = [None] * _N_WIN
    in_copies[0] = pltpu.async_copy(
        x_hbm.at[pl.ds(2 * win_start(0), 2 * _CW)], ins[0], sis[0]
    )
    for w in range(_N_WIN):
        b = w % 2
        if w + 1 < _N_WIN:
            in_copies[w + 1] = pltpu.async_copy(
                x_hbm.at[pl.ds(2 * win_start(w + 1), 2 * _CW)],
                ins[(w + 1) % 2],
                sis[(w + 1) % 2],
            )
        in_copies[w].wait()
        if w >= 2:
            out_copies[w - 2].wait()

        @plsc.parallel_loop(0, _CW // _LANES, 1, unroll=8)
        def gather_body(j, b=b, idx0=idx0):
            vals = plsc.load_gather(ins[b], [idx0 + j * (2 * _LANES)])
            outs[b][pl.ds(j * _LANES, _LANES)] = vals

        out_copies[w] = pltpu.async_copy(
            outs[b], out_hbm.at[pl.ds(win_start(w), _CW)], sos[b]
        )
    out_copies[_N_WIN - 2].wait()
    out_copies[_N_WIN - 1].wait()


@jax.jit
def _dense_dilated_sc(x_flat):
    mesh = plsc.VectorSubcoreMesh(core_axis_name="c", subcore_axis_name="s")
    return pl.kernel(
        _sc_body,
        out_type=jax.ShapeDtypeStruct((_TOTAL_OUT,), jnp.int32),
        mesh=mesh,
        scratch_types=[
            pltpu.VMEM((2 * _CW,), jnp.int32),
            pltpu.VMEM((2 * _CW,), jnp.int32),
            pltpu.VMEM((_CW,), jnp.int32),
            pltpu.VMEM((_CW,), jnp.int32),
            pltpu.SemaphoreType.DMA,
            pltpu.SemaphoreType.DMA,
            pltpu.SemaphoreType.DMA,
            pltpu.SemaphoreType.DMA,
        ],
        compiler_params=pltpu.CompilerParams(needs_layout_passes=False),
    )(x_flat)


def kernel(edge_index):
    x_flat = edge_index.reshape(-1)
    out_flat = _dense_dilated_sc(x_flat)
    return out_flat.reshape(2, 8, 10000, 9)


# speedup vs baseline: 1.5246x; 1.4585x over previous
import jax
import jax.numpy as jnp
from jax import lax
from jax.experimental import pallas as pl
from jax.experimental.pallas import tpu as pltpu
from jax.experimental.pallas import tpu_sc as plsc


def _sc_body(x_hbm, o_hbm):
    wid = lax.axis_index("s") * 2 + lax.axis_index("c")

    for k in range(18):
        @pl.when(wid == k)
        def _(k=k):
            d0, j = divmod(k, 9)
            pltpu.sync_copy(x_hbm.at[d0, 2 * j], o_hbm.at[d0, j])


@jax.jit
def _dilated_panels_sc(y):
    mesh = plsc.VectorSubcoreMesh(core_axis_name="c", subcore_axis_name="s")
    return pl.kernel(
        _sc_body,
        out_type=jax.ShapeDtypeStruct((2, 9, 8, 10000), jnp.int32),
        mesh=mesh,
        compiler_params=pltpu.CompilerParams(
            use_tc_tiling_on_sc=True, needs_layout_passes=False
        ),
    )(y)


def kernel(edge_index):
    y = jnp.transpose(edge_index, (0, 3, 1, 2))
    out_t = _dilated_panels_sc(y)
    return jnp.transpose(out_t, (0, 2, 3, 1))


# speedup vs baseline: 11.2876x; 7.4036x over previous
import jax
import jax.numpy as jnp
from jax import lax
from jax.experimental import pallas as pl
from jax.experimental.pallas import tpu as pltpu
from jax.experimental.pallas import tpu_sc as plsc


_H0 = 5120
_H1 = 10000 - _H0


def _sc_body(x_hbm, o_hbm, buf, s0, s1, s2, s3):
    wid = lax.axis_index("s") * 2 + lax.axis_index("c")

    for k in range(18):
        @pl.when(wid == k)
        def _(k=k):
            d0, j = divmod(k, 9)
            src = x_hbm.at[d0, 2 * j]
            dst = o_hbm.at[d0, j]
            c0 = pltpu.async_copy(
                src.at[:, pl.ds(0, _H0)], buf.at[:, pl.ds(0, _H0)], s0
            )
            c1 = pltpu.async_copy(
                src.at[:, pl.ds(_H0, _H1)], buf.at[:, pl.ds(_H0, _H1)], s1
            )
            c0.wait()
            o0 = pltpu.async_copy(
                buf.at[:, pl.ds(0, _H0)], dst.at[:, pl.ds(0, _H0)], s2
            )
            c1.wait()
            o1 = pltpu.async_copy(
                buf.at[:, pl.ds(_H0, _H1)], dst.at[:, pl.ds(_H0, _H1)], s3
            )
            o0.wait()
            o1.wait()


@jax.jit
def _dilated_panels_sc(y):
    mesh = plsc.VectorSubcoreMesh(core_axis_name="c", subcore_axis_name="s")
    return pl.kernel(
        _sc_body,
        out_type=jax.ShapeDtypeStruct((2, 9, 8, 10000), jnp.int32),
        mesh=mesh,
        scratch_types=[
            pltpu.VMEM((8, 10000), jnp.int32),
            pltpu.SemaphoreType.DMA,
            pltpu.SemaphoreType.DMA,
            pltpu.SemaphoreType.DMA,
            pltpu.SemaphoreType.DMA,
        ],
        compiler_params=pltpu.CompilerParams(
            use_tc_tiling_on_sc=True, needs_layout_passes=False
        ),
    )(y)


def kernel(edge_index):
    y = jnp.transpose(edge_index, (0, 3, 1, 2))
    out_t = _dilated_panels_sc(y)
    return jnp.transpose(out_t, (0, 2, 3, 1))


# speedup vs baseline: 24.1077x; 2.1358x over previous
import jax
import jax.numpy as jnp
from jax.experimental import pallas as pl
from jax.experimental.pallas import tpu as pltpu


def _tc_body(x_ref, o_ref):
    o_ref[...] = x_ref[...]


@jax.jit
def _dilated_panels_tc(y):
    return pl.pallas_call(
        _tc_body,
        grid=(2, 9),
        in_specs=[pl.BlockSpec((1, 1, 8, 10000), lambda d0, j: (d0, 2 * j, 0, 0))],
        out_specs=pl.BlockSpec((1, 1, 8, 10000), lambda d0, j: (d0, j, 0, 0)),
        out_shape=jax.ShapeDtypeStruct((2, 9, 8, 10000), jnp.int32),
        compiler_params=pltpu.CompilerParams(
            dimension_semantics=("arbitrary", "arbitrary"),
        ),
    )(y)


def kernel(edge_index):
    y = jnp.transpose(edge_index, (0, 3, 1, 2))
    out_t = _dilated_panels_tc(y)
    return jnp.transpose(out_t, (0, 2, 3, 1))
